# baseline (speedup 1.0000x reference)
import functools

import jax
import jax.numpy as jnp
from jax import lax
from jax.experimental import pallas as pl
from jax.experimental.pallas import tpu as pltpu
from jax.experimental.pallas import tpu_sc as plsc

VOCAB = 1000000
D = 128
N_IDX = 104217

NC = 2
NS = 16
NW = NC * NS

CHUNK = 128
NCHUNKS = 26
NBUF = 4
B_PER_W = CHUNK * NCHUNKS
B_PAD = B_PER_W * NW


def _gather_body(table_hbm, idx_hbm, out_hbm, idx_v,
                 rows0, rows1, rows2, rows3,
                 gs0, gs1, gs2, gs3, ss0, ss1, ss2, ss3):
    wid = lax.axis_index("s") * NC + lax.axis_index("c")
    base = wid * NCHUNKS
    bufs = (rows0, rows1, rows2, rows3)
    gsems = (gs0, gs1, gs2, gs3)
    ssems = (ss0, ss1, ss2, ss3)

    def idx_slice(i):
        return idx_v.at[pl.ds(i * CHUNK, CHUNK)]

    def out_slice(i):
        return out_hbm.at[pl.ds((base + i) * CHUNK, CHUNK)]

    pltpu.sync_copy(idx_hbm.at[pl.ds(wid * B_PER_W, B_PER_W)], idx_v)
    pltpu.async_copy(table_hbm.at[idx_slice(0)], bufs[0], gsems[0])

    for i in range(NCHUNKS):
        b = i % NBUF
        bn = (i + 1) % NBUF
        if i + 1 < NCHUNKS:
            if i >= NBUF - 1:
                pltpu.make_async_copy(
                    bufs[bn], out_slice(i - (NBUF - 1)), ssems[bn]
                ).wait()
            pltpu.async_copy(table_hbm.at[idx_slice(i + 1)], bufs[bn], gsems[bn])
        pltpu.make_async_copy(table_hbm.at[idx_slice(i)], bufs[b], gsems[b]).wait()
        pltpu.async_copy(bufs[b], out_slice(i), ssems[b])

    for i in range(NCHUNKS - NBUF, NCHUNKS):
        b = i % NBUF
        pltpu.make_async_copy(bufs[b], out_slice(i), ssems[b]).wait()


@jax.jit
def _gather(weight, idx):
    mesh = plsc.VectorSubcoreMesh(core_axis_name="c", subcore_axis_name="s")
    f = pl.kernel(
        _gather_body,
        mesh=mesh,
        out_type=jax.ShapeDtypeStruct((B_PAD, D), jnp.float32),
        scratch_types=(
            [pltpu.VMEM((B_PER_W,), jnp.int32)]
            + [pltpu.VMEM((CHUNK, D), jnp.float32)] * NBUF
            + [pltpu.SemaphoreType.DMA] * (2 * NBUF)
        ),
    )
    return f(weight, idx)


def kernel(weight, indices, offsets):
    idx = indices.astype(jnp.int32)
    idx = jnp.pad(idx, (0, B_PAD - N_IDX))
    out = _gather(weight, idx)
    return out[:N_IDX]


# speedup vs baseline: 2.1730x; 2.1730x over previous
import functools

import jax
import jax.numpy as jnp
from jax import lax
from jax.experimental import pallas as pl
from jax.experimental.pallas import tpu as pltpu
from jax.experimental.pallas import tpu_sc as plsc

VOCAB = 1000000
D = 128
N_IDX = 104217

NC = 2
NS = 16
NW = NC * NS

CHUNK = 192
NCHUNKS = 17
B_PER_W = CHUNK * NCHUNKS
B_PAD = B_PER_W * NW


def _gather_body(table_hbm, idx_hbm, out_hbm, idx_v, rows0, rows1, sem0, sem1):
    wid = lax.axis_index("s") * NC + lax.axis_index("c")
    base = wid * NCHUNKS
    bufs = (rows0, rows1)
    sems = (sem0, sem1)

    def idx_slice(i):
        return idx_v.at[pl.ds(i * CHUNK, CHUNK)]

    pltpu.sync_copy(idx_hbm.at[pl.ds(wid * B_PER_W, B_PER_W)], idx_v)
    pltpu.async_copy(table_hbm.at[idx_slice(0)], rows0, sem0)

    for i in range(NCHUNKS):
        b = i % 2
        if i + 1 < NCHUNKS:
            pltpu.async_copy(table_hbm.at[idx_slice(i + 1)], bufs[1 - b], sems[1 - b])
        pltpu.make_async_copy(table_hbm.at[idx_slice(i)], bufs[b], sems[b]).wait()
        pltpu.sync_copy(bufs[b], out_hbm.at[pl.ds((base + i) * CHUNK, CHUNK)])


@jax.jit
def _gather(weight, idx3):
    mesh = plsc.VectorSubcoreMesh(core_axis_name="c", subcore_axis_name="s")
    f = pl.kernel(
        _gather_body,
        mesh=mesh,
        out_type=jax.ShapeDtypeStruct((B_PAD, D), jnp.float32),
        scratch_types=[
            pltpu.VMEM((B_PER_W,), jnp.int32),
            pltpu.VMEM((CHUNK, D), jnp.float32),
            pltpu.VMEM((CHUNK, D), jnp.float32),
            pltpu.SemaphoreType.DMA,
            pltpu.SemaphoreType.DMA,
        ],
    )
    return f(weight, idx3)


def kernel(weight, indices, offsets):
    idx = indices.astype(jnp.int32)
    idx = jnp.pad(idx, (0, B_PAD - N_IDX))
    out = _gather(weight, idx)
    return out[:N_IDX]
